# baseline (speedup 1.0000x reference)
import functools

import jax
import jax.numpy as jnp
from jax import lax
from jax.experimental import pallas as pl
from jax.experimental.pallas import tpu as pltpu
from jax.experimental.pallas import tpu_sc as plsc

B, L, S, D, P = 128, 20, 10, 128, 16
M1 = L * S
M2 = L * S * S

LP = 2 * L
N1 = B * M1
NI = B * LP
N2 = B * M2

KPART = 4
BQ = B // KPART
NPART = BQ * M2

NC, NS = 2, 16
NW = NC * NS

GBA = 16
GBB = 4
GBC = 16


def _ring(tab_hbm, idx_v, out_hbm, base, npw, ch, bufs, sgs, sos, lag):
    nbuf = len(bufs)
    nchunk = npw // ch
    assert nchunk * ch == npw and nchunk % nbuf == 0 and 0 < lag < nbuf

    def start_gather(cc, b):
        ioff = pl.multiple_of(cc * ch, 8)
        pltpu.async_copy(
            tab_hbm.at[idx_v.at[pl.ds(ioff, ch)]], bufs[b], sgs[b])

    def wait_gather(b):
        pltpu.make_async_copy(
            tab_hbm.at[pl.ds(0, ch)], bufs[b], sgs[b]).wait()

    def start_out(cc, b):
        off = pl.multiple_of(base + cc * ch, 8)
        pltpu.async_copy(bufs[b], out_hbm.at[pl.ds(off, ch)], sos[b])

    def wait_out(b):
        pltpu.make_async_copy(
            bufs[b], out_hbm.at[pl.ds(0, ch)], sos[b]).wait()

    for c in range(lag):
        start_gather(c, c % nbuf)

    def step(i, carry):
        for b in range(nbuf):
            c = i * nbuf + b
            bn = (b + lag) % nbuf

            @pl.when(c + lag < nchunk)
            def _():
                @pl.when(c + lag >= nbuf)
                def _():
                    wait_out(bn)
                start_gather(c + lag, bn)

            wait_gather(b)
            start_out(c, b)
        return carry

    lax.fori_loop(0, nchunk // nbuf, step, 0)
    for b in range(nbuf):
        wait_out(b)


CH = 80
NBUF = 5


def _gather_quarter(idx2, table, kpart):
    mesh = plsc.VectorSubcoreMesh(core_axis_name="c", subcore_axis_name="s")
    npw = NPART // NW

    @functools.partial(
        pl.kernel,
        mesh=mesh,
        out_type=jax.ShapeDtypeStruct((NPART, D), jnp.float32),
        scratch_types=[
            pltpu.VMEM((npw,), jnp.int32),
        ] + [pltpu.VMEM((CH, D), jnp.float32)] * NBUF
          + [pltpu.SemaphoreType.DMA] * (2 * NBUF),
    )
    def k(idx_hbm, tab_hbm, out_hbm, idx_v, *rest):
        bufs = rest[:NBUF]
        sgs = rest[NBUF:2 * NBUF]
        sos = rest[2 * NBUF:]
        wid = lax.axis_index("s") * NC + lax.axis_index("c")
        base = wid * npw
        pltpu.sync_copy(idx_hbm.at[pl.ds(kpart * NPART + base, npw)], idx_v)
        _ring(tab_hbm, idx_v, out_hbm, base, npw, CH, bufs, sgs, sos, lag=3)

    return k(idx2, table)


def _gather_first(idx1, idxi, table):
    mesh = plsc.VectorSubcoreMesh(core_axis_name="c", subcore_axis_name="s")
    npw1 = N1 // NW
    npwi = NI // NW

    @functools.partial(
        pl.kernel,
        mesh=mesh,
        out_type=[
            jax.ShapeDtypeStruct((N1, D), jnp.float32),
            jax.ShapeDtypeStruct((NI, D), jnp.float32),
        ],
        scratch_types=[
            pltpu.VMEM((npw1,), jnp.int32),
            pltpu.VMEM((npwi,), jnp.int32),
        ] + [pltpu.VMEM((CH, D), jnp.float32)] * NBUF
          + [pltpu.SemaphoreType.DMA] * (2 * NBUF),
    )
    def k(idx1_hbm, idxi_hbm, tab_hbm, out1_hbm, outi_hbm,
          idx1_v, idxi_v, *rest):
        bufs = rest[:NBUF]
        sgs = rest[NBUF:2 * NBUF]
        sos = rest[2 * NBUF:]
        wid = lax.axis_index("s") * NC + lax.axis_index("c")
        pltpu.sync_copy(idx1_hbm.at[pl.ds(wid * npw1, npw1)], idx1_v)
        pltpu.sync_copy(idxi_hbm.at[pl.ds(wid * npwi, npwi)], idxi_v)
        _ring(tab_hbm, idx1_v, out1_hbm, wid * npw1, npw1, CH,
              bufs, sgs, sos, lag=3)
        _ring(tab_hbm, idxi_v, outi_hbm, wid * npwi, npwi, CH,
              bufs[:2], sgs[:2], sos[:2], lag=1)

    return k(idx1, idxi, table)


def _agg_math(neigh, self_v, wrows, pos, sess_blk, iden, w1h, w1w, w1p,
              w2m, w3a, w3b, gb, m):
    rows = gb * m * S
    groups = gb * m
    rb = m * S
    f32 = jnp.float32
    parts = []
    for g in range(gb):
        w1h_g = jnp.dot(iden * sess_blk[g:g + 1, :], w1h,
                        preferred_element_type=f32)
        parts.append(jnp.dot(neigh[g * rb:(g + 1) * rb], w1h_g,
                             preferred_element_type=f32))
    t = jnp.concatenate(parts, axis=0) if gb > 1 else parts[0]
    t = t + jnp.dot(pos, w1p, preferred_element_type=f32)
    wparts = [
        lax.dot_general(wrows[g], w1w, (((0,), (0,)), ((), ())),
                        preferred_element_type=f32)
        for g in range(gb)
    ]
    t = t + (jnp.concatenate(wparts, axis=0) if gb > 1 else wparts[0])
    t = jnp.maximum(t, 0.2 * t)
    logit = jnp.dot(t, w2m, preferred_element_type=f32)
    p = jnp.exp(logit - jnp.max(logit))
    num = jnp.sum((p * neigh).reshape(groups, S, D), axis=1)
    den = jnp.sum(p.reshape(groups, S, D), axis=1)
    agg = num / den
    out = jnp.dot(self_v, w3a, preferred_element_type=f32)
    out = out + jnp.dot(agg, w3b, preferred_element_type=f32)
    return jnp.maximum(out, 0.0)


_WSPECS = [
    pl.BlockSpec((D, D), lambda *a: (0, 0)),
    pl.BlockSpec((D, D), lambda *a: (0, 0)),
    pl.BlockSpec((1, D), lambda *a: (0, 0)),
    pl.BlockSpec((P, D), lambda *a: (0, 0)),
    pl.BlockSpec((D, D), lambda *a: (0, 0)),
    pl.BlockSpec((D, D), lambda *a: (0, 0)),
    pl.BlockSpec((D, D), lambda *a: (0, 0)),
]


def _agg_a(h, g1, items3, maskr, wrow1, pos_1, wts):
    def body(h_ref, g1_ref, it_ref, mask_ref, wr_ref, pos_ref,
             id_ref, w1h_ref, w1w_ref, w1p_ref, w2m_ref, w3a_ref, w3b_ref,
             out_ref, sess_ref):
        f32 = jnp.float32
        items = it_ref[...].reshape(GBA * LP, D)
        mask = mask_ref[0]
        gid = lax.broadcasted_iota(jnp.int32, (GBA, GBA * LP), 0)
        rid = lax.broadcasted_iota(jnp.int32, (GBA, GBA * LP), 1) // LP
        mmat = jnp.where(gid == rid, mask, 0.0)
        sess_blk = jnp.dot(mmat, items, preferred_element_type=f32)
        sess_blk = sess_blk / jnp.sum(mmat, axis=1, keepdims=True)
        sess_ref[...] = sess_blk.reshape(GBA, 1, D)
        out = _agg_math(
            g1_ref[...].reshape(GBA * M1, D),
            h_ref[...].reshape(GBA * L, D),
            wr_ref[...], pos_ref[...].reshape(GBA * M1, P), sess_blk,
            id_ref[...], w1h_ref[...], w1w_ref[...], w1p_ref[...], w2m_ref[...],
            w3a_ref[...], w3b_ref[...], GBA, L)
        out_ref[...] = out.reshape(GBA, L, D)

    return pl.pallas_call(
        body,
        grid=(B // GBA,),
        in_specs=[
            pl.BlockSpec((GBA, L, D), lambda g: (g, 0, 0)),
            pl.BlockSpec((GBA, M1, D), lambda g: (g, 0, 0)),
            pl.BlockSpec((GBA, LP, D), lambda g: (g, 0, 0)),
            pl.BlockSpec((1, 1, GBA * LP), lambda g: (g, 0, 0)),
            pl.BlockSpec((GBA, 1, M1), lambda g: (g, 0, 0)),
            pl.BlockSpec((GBA, M1, P), lambda g: (g, 0, 0)),
        ] + _WSPECS,
        out_specs=[
            pl.BlockSpec((GBA, L, D), lambda g: (g, 0, 0)),
            pl.BlockSpec((GBA, 1, D), lambda g: (g, 0, 0)),
        ],
        out_shape=[
            jax.ShapeDtypeStruct((B, L, D), jnp.float32),
            jax.ShapeDtypeStruct((B, 1, D), jnp.float32),
        ],
    )(h, g1, items3, maskr, wrow1, pos_1, *wts)


def _agg_b(kpart, g1, g2k, wrow2, pos_2, sess, wts):
    k16 = kpart * (BQ // GBB)

    def body(self_ref, neigh_ref, wr_ref, pos_ref, sess_ref,
             id_ref, w1h_ref, w1w_ref, w1p_ref, w2m_ref, w3a_ref, w3b_ref,
             out_ref):
        out = _agg_math(
            neigh_ref[...].reshape(GBB * M2, D),
            self_ref[...].reshape(GBB * M1, D),
            wr_ref[...], pos_ref[...].reshape(GBB * M2, P),
            sess_ref[...].reshape(GBB, D),
            id_ref[...], w1h_ref[...], w1w_ref[...], w1p_ref[...], w2m_ref[...],
            w3a_ref[...], w3b_ref[...], GBB, M1)
        out_ref[...] = out.reshape(GBB, M1, D)

    return pl.pallas_call(
        body,
        grid=(BQ // GBB,),
        in_specs=[
            pl.BlockSpec((GBB, M1, D), lambda g: (k16 + g, 0, 0)),
            pl.BlockSpec((GBB, M2, D), lambda g: (g, 0, 0)),
            pl.BlockSpec((GBB, 1, M2), lambda g: (k16 + g, 0, 0)),
            pl.BlockSpec((GBB, M2, P), lambda g: (k16 + g, 0, 0)),
            pl.BlockSpec((GBB, 1, D), lambda g: (k16 + g, 0, 0)),
        ] + _WSPECS,
        out_specs=pl.BlockSpec((GBB, M1, D), lambda g: (g, 0, 0)),
        out_shape=jax.ShapeDtypeStruct((BQ, M1, D), jnp.float32),
    )(g1, g2k, wrow2, pos_2, sess, *wts)


def _agg_c(kpart, out1, out2k, wrow1, pos_1, sess, wts):
    k4 = kpart * (BQ // GBC)

    def body(self_ref, neigh_ref, wr_ref, pos_ref, sess_ref,
             id_ref, w1h_ref, w1w_ref, w1p_ref, w2m_ref, w3a_ref, w3b_ref,
             out_ref):
        out = _agg_math(
            neigh_ref[...].reshape(GBC * M1, D),
            self_ref[...].reshape(GBC * L, D),
            wr_ref[...], pos_ref[...].reshape(GBC * M1, P),
            sess_ref[...].reshape(GBC, D),
            id_ref[...], w1h_ref[...], w1w_ref[...], w1p_ref[...], w2m_ref[...],
            w3a_ref[...], w3b_ref[...], GBC, L)
        out_ref[...] = out.reshape(GBC, L, D)

    return pl.pallas_call(
        body,
        grid=(BQ // GBC,),
        in_specs=[
            pl.BlockSpec((GBC, L, D), lambda g: (k4 + g, 0, 0)),
            pl.BlockSpec((GBC, M1, D), lambda g: (g, 0, 0)),
            pl.BlockSpec((GBC, 1, M1), lambda g: (k4 + g, 0, 0)),
            pl.BlockSpec((GBC, M1, P), lambda g: (k4 + g, 0, 0)),
            pl.BlockSpec((GBC, 1, D), lambda g: (k4 + g, 0, 0)),
        ] + _WSPECS,
        out_specs=pl.BlockSpec((GBC, L, D), lambda g: (g, 0, 0)),
        out_shape=jax.ShapeDtypeStruct((BQ, L, D), jnp.float32),
    )(out1, out2k, wrow1, pos_1, sess, *wts)


def kernel(h, neighbors_1, neighbors_2, weights_1, weights_2, pos_1, pos_2,
           item, mask_item, embedding, w1_0, w2_0, w3_0, w1_1, w2_1, w3_1):
    item_p = jnp.concatenate(
        [item, jnp.zeros((B, LP - L), dtype=item.dtype)], axis=1)
    idx1 = neighbors_1.reshape(-1).astype(jnp.int32)
    idxi = item_p.reshape(-1).astype(jnp.int32)
    idx2 = neighbors_2.reshape(-1).astype(jnp.int32)

    g2_0 = _gather_quarter(idx2, embedding, 0)
    g1_rows, item_rows = _gather_first(idx1, idxi, embedding)
    g2 = [g2_0.reshape(BQ, M2, D)] + [
        _gather_quarter(idx2, embedding, k).reshape(BQ, M2, D)
        for k in range(1, KPART)]
    g1 = g1_rows.reshape(B, M1, D)
    items3 = item_rows.reshape(B, LP, D)

    wrow1 = weights_1.reshape(B, 1, M1)
    wrow2 = weights_2.reshape(B, 1, M2)
    maskr = jnp.concatenate(
        [mask_item, jnp.zeros((B, LP - L), dtype=mask_item.dtype)],
        axis=1).reshape(B // GBA, 1, GBA * LP)

    iden = jnp.eye(D, dtype=jnp.float32)
    wts0 = (iden, w1_0[:D], w1_0[D:D + 1], w1_0[D + 1:], jnp.tile(w2_0, (1, D)),
            w3_0[:D], w3_0[D:])
    wts1 = (iden, w1_1[:D], w1_1[D:D + 1], w1_1[D + 1:], jnp.tile(w2_1, (1, D)),
            w3_1[:D], w3_1[D:])

    out1, sess = _agg_a(h, g1, items3, maskr, wrow1, pos_1, wts0)
    finals = []
    for k in range(KPART):
        out2k = _agg_b(k, g1, g2[k], wrow2, pos_2, sess, wts0)
        finals.append(_agg_c(k, out1, out2k, wrow1, pos_1, sess, wts1))
    return jnp.concatenate(finals, axis=0)


# speedup vs baseline: 1.0552x; 1.0552x over previous
import functools

import jax
import jax.numpy as jnp
from jax import lax
from jax.experimental import pallas as pl
from jax.experimental.pallas import tpu as pltpu
from jax.experimental.pallas import tpu_sc as plsc

B, L, S, D, P = 128, 20, 10, 128, 16
M1 = L * S
M2 = L * S * S

LP = 2 * L
N1 = B * M1
NI = B * LP
N2 = B * M2

KPART = 4
BQ = B // KPART
NPART = BQ * M2

NC, NS = 2, 16
NW = NC * NS

GBA = 16
GBB = 4
GBC = 16


def _ring(tab_hbm, idx_v, out_hbm, base, npw, ch, bufs, sgs, sos, lag):
    nbuf = len(bufs)
    nchunk = npw // ch
    assert nchunk * ch == npw and nchunk % nbuf == 0 and 0 < lag < nbuf

    def start_gather(cc, b):
        ioff = pl.multiple_of(cc * ch, 8)
        pltpu.async_copy(
            tab_hbm.at[idx_v.at[pl.ds(ioff, ch)]], bufs[b], sgs[b])

    def wait_gather(b):
        pltpu.make_async_copy(
            tab_hbm.at[pl.ds(0, ch)], bufs[b], sgs[b]).wait()

    def start_out(cc, b):
        off = pl.multiple_of(base + cc * ch, 8)
        pltpu.async_copy(bufs[b], out_hbm.at[pl.ds(off, ch)], sos[b])

    def wait_out(b):
        pltpu.make_async_copy(
            bufs[b], out_hbm.at[pl.ds(0, ch)], sos[b]).wait()

    for c in range(lag):
        start_gather(c, c % nbuf)

    def step(i, carry):
        for b in range(nbuf):
            c = i * nbuf + b
            bn = (b + lag) % nbuf

            @pl.when(c + lag < nchunk)
            def _():
                @pl.when(c + lag >= nbuf)
                def _():
                    wait_out(bn)
                start_gather(c + lag, bn)

            wait_gather(b)
            start_out(c, b)
        return carry

    lax.fori_loop(0, nchunk // nbuf, step, 0)
    for b in range(nbuf):
        wait_out(b)


CH = 80
NBUF = 5


def _gather_quarter(idx2, table, kpart):
    mesh = plsc.VectorSubcoreMesh(core_axis_name="c", subcore_axis_name="s")
    npw = NPART // NW

    @functools.partial(
        pl.kernel,
        mesh=mesh,
        out_type=jax.ShapeDtypeStruct((NPART, D), jnp.float32),
        scratch_types=[
            pltpu.VMEM((npw,), jnp.int32),
        ] + [pltpu.VMEM((CH, D), jnp.float32)] * NBUF
          + [pltpu.SemaphoreType.DMA] * (2 * NBUF),
    )
    def k(idx_hbm, tab_hbm, out_hbm, idx_v, *rest):
        bufs = rest[:NBUF]
        sgs = rest[NBUF:2 * NBUF]
        sos = rest[2 * NBUF:]
        wid = lax.axis_index("s") * NC + lax.axis_index("c")
        base = wid * npw
        pltpu.sync_copy(idx_hbm.at[pl.ds(kpart * NPART + base, npw)], idx_v)
        _ring(tab_hbm, idx_v, out_hbm, base, npw, CH, bufs, sgs, sos, lag=3)

    return k(idx2, table)


def _gather_first(idx1, idxi, table):
    mesh = plsc.VectorSubcoreMesh(core_axis_name="c", subcore_axis_name="s")
    npw1 = N1 // NW
    npwi = NI // NW

    @functools.partial(
        pl.kernel,
        mesh=mesh,
        out_type=[
            jax.ShapeDtypeStruct((N1, D), jnp.float32),
            jax.ShapeDtypeStruct((NI, D), jnp.float32),
        ],
        scratch_types=[
            pltpu.VMEM((npw1,), jnp.int32),
            pltpu.VMEM((npwi,), jnp.int32),
        ] + [pltpu.VMEM((CH, D), jnp.float32)] * NBUF
          + [pltpu.SemaphoreType.DMA] * (2 * NBUF),
    )
    def k(idx1_hbm, idxi_hbm, tab_hbm, out1_hbm, outi_hbm,
          idx1_v, idxi_v, *rest):
        bufs = rest[:NBUF]
        sgs = rest[NBUF:2 * NBUF]
        sos = rest[2 * NBUF:]
        wid = lax.axis_index("s") * NC + lax.axis_index("c")
        pltpu.sync_copy(idx1_hbm.at[pl.ds(wid * npw1, npw1)], idx1_v)
        pltpu.sync_copy(idxi_hbm.at[pl.ds(wid * npwi, npwi)], idxi_v)
        _ring(tab_hbm, idx1_v, out1_hbm, wid * npw1, npw1, CH,
              bufs, sgs, sos, lag=3)
        _ring(tab_hbm, idxi_v, outi_hbm, wid * npwi, npwi, CH,
              bufs[:2], sgs[:2], sos[:2], lag=1)

    return k(idx1, idxi, table)


def _agg_math(neigh, self_v, wrows, pos, sess_blk, iden, w1h, w1w, w1p,
              w2m, w3a, w3b, gb, m):
    rows = gb * m * S
    groups = gb * m
    rb = m * S
    f32 = jnp.float32
    parts = []
    for g in range(gb):
        w1h_g = jnp.dot(iden * sess_blk[g:g + 1, :], w1h,
                        preferred_element_type=f32)
        parts.append(jnp.dot(neigh[g * rb:(g + 1) * rb], w1h_g,
                             preferred_element_type=f32))
    t = jnp.concatenate(parts, axis=0) if gb > 1 else parts[0]
    t = t + jnp.dot(pos, w1p, preferred_element_type=f32)
    wparts = [
        lax.dot_general(wrows[g], w1w, (((0,), (0,)), ((), ())),
                        preferred_element_type=f32)
        for g in range(gb)
    ]
    t = t + (jnp.concatenate(wparts, axis=0) if gb > 1 else wparts[0])
    t = jnp.maximum(t, 0.2 * t)
    logit = jnp.dot(t, w2m, preferred_element_type=f32)
    p = jnp.exp(logit - jnp.max(logit))
    num = jnp.sum((p * neigh).reshape(groups, S, D), axis=1)
    den = jnp.sum(p.reshape(groups, S, D), axis=1)
    agg = num / den
    out = jnp.dot(self_v, w3a, preferred_element_type=f32)
    out = out + jnp.dot(agg, w3b, preferred_element_type=f32)
    return jnp.maximum(out, 0.0)


def _agg_math_mb(neighs, self_v, wrows, pos, sess_blk, iden, w1h, w1w, w1p,
                 w2m, w3a, w3b, gb, m):
    rb = m * S
    f32 = jnp.float32
    aggs = []
    for g in range(gb):
        neigh = neighs[g]
        w1h_g = jnp.dot(iden * sess_blk[g:g + 1, :], w1h,
                        preferred_element_type=f32)
        t = jnp.dot(neigh, w1h_g, preferred_element_type=f32)
        t = t + jnp.dot(pos[g * rb:(g + 1) * rb], w1p,
                        preferred_element_type=f32)
        t = t + lax.dot_general(wrows[g], w1w, (((0,), (0,)), ((), ())),
                                preferred_element_type=f32)
        t = jnp.maximum(t, 0.2 * t)
        logit = jnp.dot(t, w2m, preferred_element_type=f32)
        p = jnp.exp(logit - jnp.max(logit))
        num = jnp.sum((p * neigh).reshape(m, S, D), axis=1)
        den = jnp.sum(p.reshape(m, S, D), axis=1)
        aggs.append(num / den)
    agg = jnp.concatenate(aggs, axis=0) if gb > 1 else aggs[0]
    out = jnp.dot(self_v, w3a, preferred_element_type=f32)
    out = out + jnp.dot(agg, w3b, preferred_element_type=f32)
    return jnp.maximum(out, 0.0)


_WSPECS = [
    pl.BlockSpec((D, D), lambda *a: (0, 0)),
    pl.BlockSpec((D, D), lambda *a: (0, 0)),
    pl.BlockSpec((1, D), lambda *a: (0, 0)),
    pl.BlockSpec((P, D), lambda *a: (0, 0)),
    pl.BlockSpec((D, D), lambda *a: (0, 0)),
    pl.BlockSpec((D, D), lambda *a: (0, 0)),
    pl.BlockSpec((D, D), lambda *a: (0, 0)),
]


def _agg_a(h, g1, items3, maskr, wrow1, pos_1, wts):
    def body(h_ref, g1_ref, it_ref, mask_ref, wr_ref, pos_ref,
             id_ref, w1h_ref, w1w_ref, w1p_ref, w2m_ref, w3a_ref, w3b_ref,
             out_ref, sess_ref):
        f32 = jnp.float32
        items = it_ref[...].reshape(GBA * LP, D)
        mask = mask_ref[0]
        gid = lax.broadcasted_iota(jnp.int32, (GBA, GBA * LP), 0)
        rid = lax.broadcasted_iota(jnp.int32, (GBA, GBA * LP), 1) // LP
        mmat = jnp.where(gid == rid, mask, 0.0)
        sess_blk = jnp.dot(mmat, items, preferred_element_type=f32)
        sess_blk = sess_blk / jnp.sum(mmat, axis=1, keepdims=True)
        sess_ref[...] = sess_blk.reshape(GBA, 1, D)
        out = _agg_math(
            g1_ref[...].reshape(GBA * M1, D),
            h_ref[...].reshape(GBA * L, D),
            wr_ref[...], pos_ref[...].reshape(GBA * M1, P), sess_blk,
            id_ref[...], w1h_ref[...], w1w_ref[...], w1p_ref[...], w2m_ref[...],
            w3a_ref[...], w3b_ref[...], GBA, L)
        out_ref[...] = out.reshape(GBA, L, D)

    return pl.pallas_call(
        body,
        grid=(B // GBA,),
        in_specs=[
            pl.BlockSpec((GBA, L, D), lambda g: (g, 0, 0)),
            pl.BlockSpec((GBA, M1, D), lambda g: (g, 0, 0)),
            pl.BlockSpec((GBA, LP, D), lambda g: (g, 0, 0)),
            pl.BlockSpec((1, 1, GBA * LP), lambda g: (g, 0, 0)),
            pl.BlockSpec((GBA, 1, M1), lambda g: (g, 0, 0)),
            pl.BlockSpec((GBA, M1, P), lambda g: (g, 0, 0)),
        ] + _WSPECS,
        out_specs=[
            pl.BlockSpec((GBA, L, D), lambda g: (g, 0, 0)),
            pl.BlockSpec((GBA, 1, D), lambda g: (g, 0, 0)),
        ],
        out_shape=[
            jax.ShapeDtypeStruct((B, L, D), jnp.float32),
            jax.ShapeDtypeStruct((B, 1, D), jnp.float32),
        ],
    )(h, g1, items3, maskr, wrow1, pos_1, *wts)


def _agg_b(kpart, g1, g2k, wrow2, pos_2, sess, wts):
    k16 = kpart * (BQ // GBB)

    def body(self_ref, *refs):
        neigh_refs = refs[:GBB]
        (wr_ref, pos_ref, sess_ref, id_ref, w1h_ref, w1w_ref, w1p_ref,
         w2m_ref, w3a_ref, w3b_ref, out_ref) = refs[GBB:]
        out = _agg_math_mb(
            [r[0] for r in neigh_refs],
            self_ref[...].reshape(GBB * M1, D),
            wr_ref[...], pos_ref[...].reshape(GBB * M2, P),
            sess_ref[...].reshape(GBB, D),
            id_ref[...], w1h_ref[...], w1w_ref[...], w1p_ref[...],
            w2m_ref[...], w3a_ref[...], w3b_ref[...], GBB, M1)
        out_ref[...] = out.reshape(GBB, M1, D)

    def neigh_spec(i):
        return pl.BlockSpec((1, M2, D), lambda g: (g * GBB + i, 0, 0))

    return pl.pallas_call(
        body,
        grid=(BQ // GBB,),
        in_specs=[
            pl.BlockSpec((GBB, M1, D), lambda g: (k16 + g, 0, 0)),
        ] + [neigh_spec(i) for i in range(GBB)] + [
            pl.BlockSpec((GBB, 1, M2), lambda g: (k16 + g, 0, 0)),
            pl.BlockSpec((GBB, M2, P), lambda g: (k16 + g, 0, 0)),
            pl.BlockSpec((GBB, 1, D), lambda g: (k16 + g, 0, 0)),
        ] + _WSPECS,
        out_specs=pl.BlockSpec((GBB, M1, D), lambda g: (g, 0, 0)),
        out_shape=jax.ShapeDtypeStruct((BQ, M1, D), jnp.float32),
    )(g1, *([g2k] * GBB), wrow2, pos_2, sess, *wts)


def _agg_c(kpart, out1, out2k, wrow1, pos_1, sess, wts):
    k4 = kpart * (BQ // GBC)

    def body(self_ref, neigh_ref, wr_ref, pos_ref, sess_ref,
             id_ref, w1h_ref, w1w_ref, w1p_ref, w2m_ref, w3a_ref, w3b_ref,
             out_ref):
        out = _agg_math(
            neigh_ref[...].reshape(GBC * M1, D),
            self_ref[...].reshape(GBC * L, D),
            wr_ref[...], pos_ref[...].reshape(GBC * M1, P),
            sess_ref[...].reshape(GBC, D),
            id_ref[...], w1h_ref[...], w1w_ref[...], w1p_ref[...], w2m_ref[...],
            w3a_ref[...], w3b_ref[...], GBC, L)
        out_ref[...] = out.reshape(GBC, L, D)

    return pl.pallas_call(
        body,
        grid=(BQ // GBC,),
        in_specs=[
            pl.BlockSpec((GBC, L, D), lambda g: (k4 + g, 0, 0)),
            pl.BlockSpec((GBC, M1, D), lambda g: (g, 0, 0)),
            pl.BlockSpec((GBC, 1, M1), lambda g: (k4 + g, 0, 0)),
            pl.BlockSpec((GBC, M1, P), lambda g: (k4 + g, 0, 0)),
            pl.BlockSpec((GBC, 1, D), lambda g: (k4 + g, 0, 0)),
        ] + _WSPECS,
        out_specs=pl.BlockSpec((GBC, L, D), lambda g: (g, 0, 0)),
        out_shape=jax.ShapeDtypeStruct((BQ, L, D), jnp.float32),
    )(out1, out2k, wrow1, pos_1, sess, *wts)


def kernel(h, neighbors_1, neighbors_2, weights_1, weights_2, pos_1, pos_2,
           item, mask_item, embedding, w1_0, w2_0, w3_0, w1_1, w2_1, w3_1):
    item_p = jnp.concatenate(
        [item, jnp.zeros((B, LP - L), dtype=item.dtype)], axis=1)
    idx1 = neighbors_1.reshape(-1).astype(jnp.int32)
    idxi = item_p.reshape(-1).astype(jnp.int32)
    idx2 = neighbors_2.reshape(-1).astype(jnp.int32)

    g2_0 = _gather_quarter(idx2, embedding, 0)
    g1_rows, item_rows = _gather_first(idx1, idxi, embedding)
    g2 = [g2_0.reshape(BQ, M2, D)] + [
        _gather_quarter(idx2, embedding, k).reshape(BQ, M2, D)
        for k in range(1, KPART)]
    g1 = g1_rows.reshape(B, M1, D)
    items3 = item_rows.reshape(B, LP, D)

    wrow1 = weights_1.reshape(B, 1, M1)
    wrow2 = weights_2.reshape(B, 1, M2)
    maskr = jnp.concatenate(
        [mask_item, jnp.zeros((B, LP - L), dtype=mask_item.dtype)],
        axis=1).reshape(B // GBA, 1, GBA * LP)

    iden = jnp.eye(D, dtype=jnp.float32)
    wts0 = (iden, w1_0[:D], w1_0[D:D + 1], w1_0[D + 1:], jnp.tile(w2_0, (1, D)),
            w3_0[:D], w3_0[D:])
    wts1 = (iden, w1_1[:D], w1_1[D:D + 1], w1_1[D + 1:], jnp.tile(w2_1, (1, D)),
            w3_1[:D], w3_1[D:])

    out1, sess = _agg_a(h, g1, items3, maskr, wrow1, pos_1, wts0)
    finals = []
    for k in range(KPART):
        out2k = _agg_b(k, g1, g2[k], wrow2, pos_2, sess, wts0)
        finals.append(_agg_c(k, out1, out2k, wrow1, pos_1, sess, wts1))
    return jnp.concatenate(finals, axis=0)


# speedup vs baseline: 1.0741x; 1.0179x over previous
import functools

import jax
import jax.numpy as jnp
from jax import lax
from jax.experimental import pallas as pl
from jax.experimental.pallas import tpu as pltpu
from jax.experimental.pallas import tpu_sc as plsc

B, L, S, D, P = 128, 20, 10, 128, 16
M1 = L * S
M2 = L * S * S

LP = 2 * L
N1 = B * M1
NI = B * LP
N2 = B * M2

KPART = 4
BQ = B // KPART
NPART = BQ * M2

NC, NS = 2, 16
NW = NC * NS

GBA = 16
GBB = 4
GBC = 16


def _ring(tab_hbm, idx_v, out_hbm, base, npw, ch, bufs, sgs, sos, lag):
    nbuf = len(bufs)
    nchunk = npw // ch
    assert nchunk * ch == npw and nchunk % nbuf == 0 and 0 < lag < nbuf

    def start_gather(cc, b):
        ioff = pl.multiple_of(cc * ch, 8)
        pltpu.async_copy(
            tab_hbm.at[idx_v.at[pl.ds(ioff, ch)]], bufs[b], sgs[b])

    def wait_gather(b):
        pltpu.make_async_copy(
            tab_hbm.at[pl.ds(0, ch)], bufs[b], sgs[b]).wait()

    def start_out(cc, b):
        off = pl.multiple_of(base + cc * ch, 8)
        pltpu.async_copy(bufs[b], out_hbm.at[pl.ds(off, ch)], sos[b])

    def wait_out(b):
        pltpu.make_async_copy(
            bufs[b], out_hbm.at[pl.ds(0, ch)], sos[b]).wait()

    for c in range(lag):
        start_gather(c, c % nbuf)

    def step(i, carry):
        for b in range(nbuf):
            c = i * nbuf + b
            bn = (b + lag) % nbuf

            @pl.when(c + lag < nchunk)
            def _():
                @pl.when(c + lag >= nbuf)
                def _():
                    wait_out(bn)
                start_gather(c + lag, bn)

            wait_gather(b)
            start_out(c, b)
        return carry

    lax.fori_loop(0, nchunk // nbuf, step, 0)
    for b in range(nbuf):
        wait_out(b)


CH = 80
NBUF = 5


def _gather_quarter(idx2, table, kpart):
    mesh = plsc.VectorSubcoreMesh(core_axis_name="c", subcore_axis_name="s")
    npw = NPART // NW

    @functools.partial(
        pl.kernel,
        mesh=mesh,
        out_type=jax.ShapeDtypeStruct((NPART, D), jnp.float32),
        scratch_types=[
            pltpu.VMEM((npw,), jnp.int32),
        ] + [pltpu.VMEM((CH, D), jnp.float32)] * NBUF
          + [pltpu.SemaphoreType.DMA] * (2 * NBUF),
    )
    def k(idx_hbm, tab_hbm, out_hbm, idx_v, *rest):
        bufs = rest[:NBUF]
        sgs = rest[NBUF:2 * NBUF]
        sos = rest[2 * NBUF:]
        wid = lax.axis_index("s") * NC + lax.axis_index("c")
        base = wid * npw
        pltpu.sync_copy(idx_hbm.at[pl.ds(kpart * NPART + base, npw)], idx_v)
        _ring(tab_hbm, idx_v, out_hbm, base, npw, CH, bufs, sgs, sos, lag=3)

    return k(idx2, table)


def _gather_first(idx1, idxi, table):
    mesh = plsc.VectorSubcoreMesh(core_axis_name="c", subcore_axis_name="s")
    npw1 = N1 // NW
    npwi = NI // NW

    @functools.partial(
        pl.kernel,
        mesh=mesh,
        out_type=[
            jax.ShapeDtypeStruct((N1, D), jnp.float32),
            jax.ShapeDtypeStruct((NI, D), jnp.float32),
        ],
        scratch_types=[
            pltpu.VMEM((npw1,), jnp.int32),
            pltpu.VMEM((npwi,), jnp.int32),
        ] + [pltpu.VMEM((CH, D), jnp.float32)] * NBUF
          + [pltpu.SemaphoreType.DMA] * (2 * NBUF),
    )
    def k(idx1_hbm, idxi_hbm, tab_hbm, out1_hbm, outi_hbm,
          idx1_v, idxi_v, *rest):
        bufs = rest[:NBUF]
        sgs = rest[NBUF:2 * NBUF]
        sos = rest[2 * NBUF:]
        wid = lax.axis_index("s") * NC + lax.axis_index("c")
        pltpu.sync_copy(idx1_hbm.at[pl.ds(wid * npw1, npw1)], idx1_v)
        pltpu.sync_copy(idxi_hbm.at[pl.ds(wid * npwi, npwi)], idxi_v)
        _ring(tab_hbm, idx1_v, out1_hbm, wid * npw1, npw1, CH,
              bufs, sgs, sos, lag=3)
        _ring(tab_hbm, idxi_v, outi_hbm, wid * npwi, npwi, CH,
              bufs[:2], sgs[:2], sos[:2], lag=1)

    return k(idx1, idxi, table)


def _agg_math(neigh, self_v, wrows, pos, sess_blk, iden, w1h, w1w, w1p,
              w2m, w3a, w3b, gb, m):
    rows = gb * m * S
    groups = gb * m
    rb = m * S
    f32 = jnp.float32
    parts = []
    for g in range(gb):
        w1h_g = jnp.dot(iden * sess_blk[g:g + 1, :], w1h,
                        preferred_element_type=f32)
        parts.append(jnp.dot(neigh[g * rb:(g + 1) * rb], w1h_g,
                             preferred_element_type=f32))
    t = jnp.concatenate(parts, axis=0) if gb > 1 else parts[0]
    t = t + jnp.dot(pos, w1p, preferred_element_type=f32)
    wparts = [
        lax.dot_general(wrows[g], w1w, (((0,), (0,)), ((), ())),
                        preferred_element_type=f32)
        for g in range(gb)
    ]
    t = t + (jnp.concatenate(wparts, axis=0) if gb > 1 else wparts[0])
    t = jnp.maximum(t, 0.2 * t)
    logit = jnp.dot(t, w2m, preferred_element_type=f32)
    p = jnp.exp(logit - jnp.max(logit))
    num = jnp.sum((p * neigh).reshape(groups, S, D), axis=1)
    den = jnp.sum(p.reshape(groups, S, D), axis=1)
    agg = num / den
    out = jnp.dot(self_v, w3a, preferred_element_type=f32)
    out = out + jnp.dot(agg, w3b, preferred_element_type=f32)
    return jnp.maximum(out, 0.0)


def _agg_math_mb(neighs, self_v, wrows, pos, sess_blk, iden, w1h, w1w, w1p,
                 w2m, w3a, w3b, gb, m):
    rb = m * S
    f32 = jnp.float32
    aggs = []
    for g in range(gb):
        neigh = neighs[g]
        w1h_g = jnp.dot(iden * sess_blk[g:g + 1, :], w1h,
                        preferred_element_type=f32)
        t = jnp.dot(neigh, w1h_g, preferred_element_type=f32)
        t = t + jnp.dot(pos[g * rb:(g + 1) * rb], w1p,
                        preferred_element_type=f32)
        t = t + lax.dot_general(wrows[g], w1w, (((0,), (0,)), ((), ())),
                                preferred_element_type=f32)
        t = jnp.maximum(t, 0.2 * t)
        logit = jnp.dot(t, w2m, preferred_element_type=f32)
        p = jnp.exp(logit - jnp.max(logit))
        num = jnp.sum((p * neigh).reshape(m, S, D), axis=1)
        den = jnp.sum(p.reshape(m, S, D), axis=1)
        aggs.append(num / den)
    agg = jnp.concatenate(aggs, axis=0) if gb > 1 else aggs[0]
    out = jnp.dot(self_v, w3a, preferred_element_type=f32)
    out = out + jnp.dot(agg, w3b, preferred_element_type=f32)
    return jnp.maximum(out, 0.0)


_WSPECS = [
    pl.BlockSpec((D, D), lambda *a: (0, 0)),
    pl.BlockSpec((D, D), lambda *a: (0, 0)),
    pl.BlockSpec((1, D), lambda *a: (0, 0)),
    pl.BlockSpec((P, D), lambda *a: (0, 0)),
    pl.BlockSpec((D, D), lambda *a: (0, 0)),
    pl.BlockSpec((D, D), lambda *a: (0, 0)),
    pl.BlockSpec((D, D), lambda *a: (0, 0)),
]


def _agg_a(h, g1, items3, maskr, wrow1, pos_1, wts):
    def body(h_ref, g1_ref, it_ref, mask_ref, wr_ref, pos_ref,
             id_ref, w1h_ref, w1w_ref, w1p_ref, w2m_ref, w3a_ref, w3b_ref,
             out_ref, sess_ref):
        f32 = jnp.float32
        items = it_ref[...].reshape(GBA * LP, D)
        mask = mask_ref[0]
        gid = lax.broadcasted_iota(jnp.int32, (GBA, GBA * LP), 0)
        rid = lax.broadcasted_iota(jnp.int32, (GBA, GBA * LP), 1) // LP
        mmat = jnp.where(gid == rid, mask, 0.0)
        sess_blk = jnp.dot(mmat, items, preferred_element_type=f32)
        sess_blk = sess_blk / jnp.sum(mmat, axis=1, keepdims=True)
        sess_ref[...] = sess_blk.reshape(GBA, 1, D)
        out = _agg_math(
            g1_ref[...].reshape(GBA * M1, D),
            h_ref[...].reshape(GBA * L, D),
            wr_ref[...], pos_ref[...].reshape(GBA * M1, P), sess_blk,
            id_ref[...], w1h_ref[...], w1w_ref[...], w1p_ref[...], w2m_ref[...],
            w3a_ref[...], w3b_ref[...], GBA, L)
        out_ref[...] = out.reshape(GBA, L, D)

    return pl.pallas_call(
        body,
        grid=(B // GBA,),
        in_specs=[
            pl.BlockSpec((GBA, L, D), lambda g: (g, 0, 0)),
            pl.BlockSpec((GBA, M1, D), lambda g: (g, 0, 0)),
            pl.BlockSpec((GBA, LP, D), lambda g: (g, 0, 0)),
            pl.BlockSpec((1, 1, GBA * LP), lambda g: (g, 0, 0)),
            pl.BlockSpec((GBA, 1, M1), lambda g: (g, 0, 0)),
            pl.BlockSpec((GBA, M1, P), lambda g: (g, 0, 0)),
        ] + _WSPECS,
        out_specs=[
            pl.BlockSpec((GBA, L, D), lambda g: (g, 0, 0)),
            pl.BlockSpec((GBA, 1, D), lambda g: (g, 0, 0)),
        ],
        out_shape=[
            jax.ShapeDtypeStruct((B, L, D), jnp.float32),
            jax.ShapeDtypeStruct((B, 1, D), jnp.float32),
        ],
    )(h, g1, items3, maskr, wrow1, pos_1, *wts)


def _agg_bc(kpart, g1, g2k, wrow2, pos_2, sess, out1, wrow1, pos_1,
            wts0, wts1):
    k16 = kpart * (BQ // GBB)

    def body(self_ref, *refs):
        neigh_refs = refs[:GBB]
        (wr2_ref, pos2_ref, sess_ref, o1_ref, wr1_ref, pos1_ref,
         id_ref, w1h0, w1w0, w1p0, w2m0, w3a0, w3b0,
         w1h1, w1w1, w1p1, w2m1, w3a1, w3b1, out_ref) = refs[GBB:]
        sess_blk = sess_ref[...].reshape(GBB, D)
        out2 = _agg_math_mb(
            [r[0] for r in neigh_refs],
            self_ref[...].reshape(GBB * M1, D),
            wr2_ref[...], pos2_ref[...].reshape(GBB * M2, P), sess_blk,
            id_ref[...], w1h0[...], w1w0[...], w1p0[...],
            w2m0[...], w3a0[...], w3b0[...], GBB, M1)
        final = _agg_math_mb(
            [out2[g * M1:(g + 1) * M1] for g in range(GBB)],
            o1_ref[...].reshape(GBB * L, D),
            wr1_ref[...], pos1_ref[...].reshape(GBB * M1, P), sess_blk,
            id_ref[...], w1h1[...], w1w1[...], w1p1[...],
            w2m1[...], w3a1[...], w3b1[...], GBB, L)
        out_ref[...] = final.reshape(GBB, L, D)

    def neigh_spec(i):
        return pl.BlockSpec((1, M2, D), lambda g: (g * GBB + i, 0, 0))

    return pl.pallas_call(
        body,
        grid=(BQ // GBB,),
        in_specs=[
            pl.BlockSpec((GBB, M1, D), lambda g: (k16 + g, 0, 0)),
        ] + [neigh_spec(i) for i in range(GBB)] + [
            pl.BlockSpec((GBB, 1, M2), lambda g: (k16 + g, 0, 0)),
            pl.BlockSpec((GBB, M2, P), lambda g: (k16 + g, 0, 0)),
            pl.BlockSpec((GBB, 1, D), lambda g: (k16 + g, 0, 0)),
            pl.BlockSpec((GBB, L, D), lambda g: (k16 + g, 0, 0)),
            pl.BlockSpec((GBB, 1, M1), lambda g: (k16 + g, 0, 0)),
            pl.BlockSpec((GBB, M1, P), lambda g: (k16 + g, 0, 0)),
        ] + _WSPECS + _WSPECS[1:],
        out_specs=pl.BlockSpec((GBB, L, D), lambda g: (g, 0, 0)),
        out_shape=jax.ShapeDtypeStruct((BQ, L, D), jnp.float32),
    )(g1, *([g2k] * GBB), wrow2, pos_2, sess, out1, wrow1, pos_1,
      *wts0, *wts1[1:])


def kernel(h, neighbors_1, neighbors_2, weights_1, weights_2, pos_1, pos_2,
           item, mask_item, embedding, w1_0, w2_0, w3_0, w1_1, w2_1, w3_1):
    item_p = jnp.concatenate(
        [item, jnp.zeros((B, LP - L), dtype=item.dtype)], axis=1)
    idx1 = neighbors_1.reshape(-1).astype(jnp.int32)
    idxi = item_p.reshape(-1).astype(jnp.int32)
    idx2 = neighbors_2.reshape(-1).astype(jnp.int32)

    g2_0 = _gather_quarter(idx2, embedding, 0)
    g1_rows, item_rows = _gather_first(idx1, idxi, embedding)
    g2 = [g2_0.reshape(BQ, M2, D)] + [
        _gather_quarter(idx2, embedding, k).reshape(BQ, M2, D)
        for k in range(1, KPART)]
    g1 = g1_rows.reshape(B, M1, D)
    items3 = item_rows.reshape(B, LP, D)

    wrow1 = weights_1.reshape(B, 1, M1)
    wrow2 = weights_2.reshape(B, 1, M2)
    maskr = jnp.concatenate(
        [mask_item, jnp.zeros((B, LP - L), dtype=mask_item.dtype)],
        axis=1).reshape(B // GBA, 1, GBA * LP)

    iden = jnp.eye(D, dtype=jnp.float32)
    wts0 = (iden, w1_0[:D], w1_0[D:D + 1], w1_0[D + 1:], jnp.tile(w2_0, (1, D)),
            w3_0[:D], w3_0[D:])
    wts1 = (iden, w1_1[:D], w1_1[D:D + 1], w1_1[D + 1:], jnp.tile(w2_1, (1, D)),
            w3_1[:D], w3_1[D:])

    out1, sess = _agg_a(h, g1, items3, maskr, wrow1, pos_1, wts0)
    finals = [
        _agg_bc(k, g1, g2[k], wrow2, pos_2, sess, out1, wrow1, pos_1,
                wts0, wts1)
        for k in range(KPART)]
    return jnp.concatenate(finals, axis=0)


# speedup vs baseline: 1.0824x; 1.0078x over previous
import functools

import jax
import jax.numpy as jnp
from jax import lax
from jax.experimental import pallas as pl
from jax.experimental.pallas import tpu as pltpu
from jax.experimental.pallas import tpu_sc as plsc

B, L, S, D, P = 128, 20, 10, 128, 16
M1 = L * S
M2 = L * S * S

LP = 2 * L
N1 = B * M1
NI = B * LP
N2 = B * M2

KPART = 4
BQ = B // KPART
NPART = BQ * M2

NC, NS = 2, 16
NW = NC * NS

GBA = 16
GBB = 8
GBC = 16


def _ring(tab_hbm, idx_v, out_hbm, base, npw, ch, bufs, sgs, sos, lag):
    nbuf = len(bufs)
    nchunk = npw // ch
    assert nchunk * ch == npw and nchunk % nbuf == 0 and 0 < lag < nbuf

    def start_gather(cc, b):
        ioff = pl.multiple_of(cc * ch, 8)
        pltpu.async_copy(
            tab_hbm.at[idx_v.at[pl.ds(ioff, ch)]], bufs[b], sgs[b])

    def wait_gather(b):
        pltpu.make_async_copy(
            tab_hbm.at[pl.ds(0, ch)], bufs[b], sgs[b]).wait()

    def start_out(cc, b):
        off = pl.multiple_of(base + cc * ch, 8)
        pltpu.async_copy(bufs[b], out_hbm.at[pl.ds(off, ch)], sos[b])

    def wait_out(b):
        pltpu.make_async_copy(
            bufs[b], out_hbm.at[pl.ds(0, ch)], sos[b]).wait()

    for c in range(lag):
        start_gather(c, c % nbuf)

    def step(i, carry):
        for b in range(nbuf):
            c = i * nbuf + b
            bn = (b + lag) % nbuf

            @pl.when(c + lag < nchunk)
            def _():
                @pl.when(c + lag >= nbuf)
                def _():
                    wait_out(bn)
                start_gather(c + lag, bn)

            wait_gather(b)
            start_out(c, b)
        return carry

    lax.fori_loop(0, nchunk // nbuf, step, 0)
    for b in range(nbuf):
        wait_out(b)


CH = 80
NBUF = 5


def _gather_quarter(idx2, table, kpart):
    mesh = plsc.VectorSubcoreMesh(core_axis_name="c", subcore_axis_name="s")
    npw = NPART // NW

    @functools.partial(
        pl.kernel,
        mesh=mesh,
        out_type=jax.ShapeDtypeStruct((NPART, D), jnp.float32),
        scratch_types=[
            pltpu.VMEM((npw,), jnp.int32),
        ] + [pltpu.VMEM((CH, D), jnp.float32)] * NBUF
          + [pltpu.SemaphoreType.DMA] * (2 * NBUF),
    )
    def k(idx_hbm, tab_hbm, out_hbm, idx_v, *rest):
        bufs = rest[:NBUF]
        sgs = rest[NBUF:2 * NBUF]
        sos = rest[2 * NBUF:]
        wid = lax.axis_index("s") * NC + lax.axis_index("c")
        base = wid * npw
        pltpu.sync_copy(idx_hbm.at[pl.ds(kpart * NPART + base, npw)], idx_v)
        _ring(tab_hbm, idx_v, out_hbm, base, npw, CH, bufs, sgs, sos, lag=3)

    return k(idx2, table)


def _gather_first(idx1, idxi, table):
    mesh = plsc.VectorSubcoreMesh(core_axis_name="c", subcore_axis_name="s")
    npw1 = N1 // NW
    npwi = NI // NW

    @functools.partial(
        pl.kernel,
        mesh=mesh,
        out_type=[
            jax.ShapeDtypeStruct((N1, D), jnp.float32),
            jax.ShapeDtypeStruct((NI, D), jnp.float32),
        ],
        scratch_types=[
            pltpu.VMEM((npw1,), jnp.int32),
            pltpu.VMEM((npwi,), jnp.int32),
        ] + [pltpu.VMEM((CH, D), jnp.float32)] * NBUF
          + [pltpu.SemaphoreType.DMA] * (2 * NBUF),
    )
    def k(idx1_hbm, idxi_hbm, tab_hbm, out1_hbm, outi_hbm,
          idx1_v, idxi_v, *rest):
        bufs = rest[:NBUF]
        sgs = rest[NBUF:2 * NBUF]
        sos = rest[2 * NBUF:]
        wid = lax.axis_index("s") * NC + lax.axis_index("c")
        pltpu.sync_copy(idx1_hbm.at[pl.ds(wid * npw1, npw1)], idx1_v)
        pltpu.sync_copy(idxi_hbm.at[pl.ds(wid * npwi, npwi)], idxi_v)
        _ring(tab_hbm, idx1_v, out1_hbm, wid * npw1, npw1, CH,
              bufs, sgs, sos, lag=3)
        _ring(tab_hbm, idxi_v, outi_hbm, wid * npwi, npwi, CH,
              bufs[:2], sgs[:2], sos[:2], lag=1)

    return k(idx1, idxi, table)


def _agg_math(neigh, self_v, wrows, pos, sess_blk, iden, w1h, w1w, w1p,
              w2m, w3a, w3b, gb, m):
    rows = gb * m * S
    groups = gb * m
    rb = m * S
    f32 = jnp.float32
    parts = []
    for g in range(gb):
        w1h_g = jnp.dot(iden * sess_blk[g:g + 1, :], w1h,
                        preferred_element_type=f32)
        parts.append(jnp.dot(neigh[g * rb:(g + 1) * rb], w1h_g,
                             preferred_element_type=f32))
    t = jnp.concatenate(parts, axis=0) if gb > 1 else parts[0]
    t = t + jnp.dot(pos, w1p, preferred_element_type=f32)
    wparts = [
        lax.dot_general(wrows[g], w1w, (((0,), (0,)), ((), ())),
                        preferred_element_type=f32)
        for g in range(gb)
    ]
    t = t + (jnp.concatenate(wparts, axis=0) if gb > 1 else wparts[0])
    t = jnp.maximum(t, 0.2 * t)
    logit = jnp.dot(t, w2m, preferred_element_type=f32)
    p = jnp.exp(logit - jnp.max(logit))
    num = jnp.sum((p * neigh).reshape(groups, S, D), axis=1)
    den = jnp.sum(p.reshape(groups, S, D), axis=1)
    agg = num / den
    out = jnp.dot(self_v, w3a, preferred_element_type=f32)
    out = out + jnp.dot(agg, w3b, preferred_element_type=f32)
    return jnp.maximum(out, 0.0)


def _agg_math_mb(neighs, self_v, wrows, pos, sess_blk, iden, w1h, w1w, w1p,
                 w2m, w3a, w3b, gb, m):
    rb = m * S
    f32 = jnp.float32
    aggs = []
    for g in range(gb):
        neigh = neighs[g]
        w1h_g = jnp.dot(iden * sess_blk[g:g + 1, :], w1h,
                        preferred_element_type=f32)
        t = jnp.dot(neigh, w1h_g, preferred_element_type=f32)
        t = t + jnp.dot(pos[g * rb:(g + 1) * rb], w1p,
                        preferred_element_type=f32)
        t = t + lax.dot_general(wrows[g], w1w, (((0,), (0,)), ((), ())),
                                preferred_element_type=f32)
        t = jnp.maximum(t, 0.2 * t)
        logit = jnp.dot(t, w2m, preferred_element_type=f32)
        p = jnp.exp(logit - jnp.max(logit))
        num = jnp.sum((p * neigh).reshape(m, S, D), axis=1)
        den = jnp.sum(p.reshape(m, S, D), axis=1)
        aggs.append(num / den)
    agg = jnp.concatenate(aggs, axis=0) if gb > 1 else aggs[0]
    out = jnp.dot(self_v, w3a, preferred_element_type=f32)
    out = out + jnp.dot(agg, w3b, preferred_element_type=f32)
    return jnp.maximum(out, 0.0)


_WSPECS = [
    pl.BlockSpec((D, D), lambda *a: (0, 0)),
    pl.BlockSpec((D, D), lambda *a: (0, 0)),
    pl.BlockSpec((1, D), lambda *a: (0, 0)),
    pl.BlockSpec((P, D), lambda *a: (0, 0)),
    pl.BlockSpec((D, D), lambda *a: (0, 0)),
    pl.BlockSpec((D, D), lambda *a: (0, 0)),
    pl.BlockSpec((D, D), lambda *a: (0, 0)),
]


def _agg_a(h, g1, items3, maskr, wrow1, pos_1, wts):
    def body(h_ref, g1_ref, it_ref, mask_ref, wr_ref, pos_ref,
             id_ref, w1h_ref, w1w_ref, w1p_ref, w2m_ref, w3a_ref, w3b_ref,
             out_ref, sess_ref):
        f32 = jnp.float32
        items = it_ref[...].reshape(GBA * LP, D)
        mask = mask_ref[0]
        gid = lax.broadcasted_iota(jnp.int32, (GBA, GBA * LP), 0)
        rid = lax.broadcasted_iota(jnp.int32, (GBA, GBA * LP), 1) // LP
        mmat = jnp.where(gid == rid, mask, 0.0)
        sess_blk = jnp.dot(mmat, items, preferred_element_type=f32)
        sess_blk = sess_blk / jnp.sum(mmat, axis=1, keepdims=True)
        sess_ref[...] = sess_blk.reshape(GBA, 1, D)
        out = _agg_math(
            g1_ref[...].reshape(GBA * M1, D),
            h_ref[...].reshape(GBA * L, D),
            wr_ref[...], pos_ref[...].reshape(GBA * M1, P), sess_blk,
            id_ref[...], w1h_ref[...], w1w_ref[...], w1p_ref[...], w2m_ref[...],
            w3a_ref[...], w3b_ref[...], GBA, L)
        out_ref[...] = out.reshape(GBA, L, D)

    return pl.pallas_call(
        body,
        grid=(B // GBA,),
        in_specs=[
            pl.BlockSpec((GBA, L, D), lambda g: (g, 0, 0)),
            pl.BlockSpec((GBA, M1, D), lambda g: (g, 0, 0)),
            pl.BlockSpec((GBA, LP, D), lambda g: (g, 0, 0)),
            pl.BlockSpec((1, 1, GBA * LP), lambda g: (g, 0, 0)),
            pl.BlockSpec((GBA, 1, M1), lambda g: (g, 0, 0)),
            pl.BlockSpec((GBA, M1, P), lambda g: (g, 0, 0)),
        ] + _WSPECS,
        out_specs=[
            pl.BlockSpec((GBA, L, D), lambda g: (g, 0, 0)),
            pl.BlockSpec((GBA, 1, D), lambda g: (g, 0, 0)),
        ],
        out_shape=[
            jax.ShapeDtypeStruct((B, L, D), jnp.float32),
            jax.ShapeDtypeStruct((B, 1, D), jnp.float32),
        ],
    )(h, g1, items3, maskr, wrow1, pos_1, *wts)


def _agg_bc(kpart, g1, g2k, wrow2, pos_2, sess, out1, wrow1, pos_1,
            wts0, wts1):
    k16 = kpart * (BQ // GBB)

    def body(self_ref, *refs):
        neigh_refs = refs[:GBB]
        (wr2_ref, pos2_ref, sess_ref, o1_ref, wr1_ref, pos1_ref,
         id_ref, w1h0, w1w0, w1p0, w2m0, w3a0, w3b0,
         w1h1, w1w1, w1p1, w2m1, w3a1, w3b1, out_ref) = refs[GBB:]
        sess_blk = sess_ref[...].reshape(GBB, D)
        out2 = _agg_math_mb(
            [r[0] for r in neigh_refs],
            self_ref[...].reshape(GBB * M1, D),
            wr2_ref[...], pos2_ref[...].reshape(GBB * M2, P), sess_blk,
            id_ref[...], w1h0[...], w1w0[...], w1p0[...],
            w2m0[...], w3a0[...], w3b0[...], GBB, M1)
        final = _agg_math_mb(
            [out2[g * M1:(g + 1) * M1] for g in range(GBB)],
            o1_ref[...].reshape(GBB * L, D),
            wr1_ref[...], pos1_ref[...].reshape(GBB * M1, P), sess_blk,
            id_ref[...], w1h1[...], w1w1[...], w1p1[...],
            w2m1[...], w3a1[...], w3b1[...], GBB, L)
        out_ref[...] = final.reshape(GBB, L, D)

    def neigh_spec(i):
        return pl.BlockSpec((1, M2, D), lambda g: (g * GBB + i, 0, 0))

    return pl.pallas_call(
        body,
        grid=(BQ // GBB,),
        in_specs=[
            pl.BlockSpec((GBB, M1, D), lambda g: (k16 + g, 0, 0)),
        ] + [neigh_spec(i) for i in range(GBB)] + [
            pl.BlockSpec((GBB, 1, M2), lambda g: (k16 + g, 0, 0)),
            pl.BlockSpec((GBB, M2, P), lambda g: (k16 + g, 0, 0)),
            pl.BlockSpec((GBB, 1, D), lambda g: (k16 + g, 0, 0)),
            pl.BlockSpec((GBB, L, D), lambda g: (k16 + g, 0, 0)),
            pl.BlockSpec((GBB, 1, M1), lambda g: (k16 + g, 0, 0)),
            pl.BlockSpec((GBB, M1, P), lambda g: (k16 + g, 0, 0)),
        ] + _WSPECS + _WSPECS[1:],
        out_specs=pl.BlockSpec((GBB, L, D), lambda g: (g, 0, 0)),
        out_shape=jax.ShapeDtypeStruct((BQ, L, D), jnp.float32),
    )(g1, *([g2k] * GBB), wrow2, pos_2, sess, out1, wrow1, pos_1,
      *wts0, *wts1[1:])


def kernel(h, neighbors_1, neighbors_2, weights_1, weights_2, pos_1, pos_2,
           item, mask_item, embedding, w1_0, w2_0, w3_0, w1_1, w2_1, w3_1):
    item_p = jnp.concatenate(
        [item, jnp.zeros((B, LP - L), dtype=item.dtype)], axis=1)
    idx1 = neighbors_1.reshape(-1).astype(jnp.int32)
    idxi = item_p.reshape(-1).astype(jnp.int32)
    idx2 = neighbors_2.reshape(-1).astype(jnp.int32)

    g2_0 = _gather_quarter(idx2, embedding, 0)
    g1_rows, item_rows = _gather_first(idx1, idxi, embedding)
    g2 = [g2_0.reshape(BQ, M2, D)] + [
        _gather_quarter(idx2, embedding, k).reshape(BQ, M2, D)
        for k in range(1, KPART)]
    g1 = g1_rows.reshape(B, M1, D)
    items3 = item_rows.reshape(B, LP, D)

    wrow1 = weights_1.reshape(B, 1, M1)
    wrow2 = weights_2.reshape(B, 1, M2)
    maskr = jnp.concatenate(
        [mask_item, jnp.zeros((B, LP - L), dtype=mask_item.dtype)],
        axis=1).reshape(B // GBA, 1, GBA * LP)

    iden = jnp.eye(D, dtype=jnp.float32)
    wts0 = (iden, w1_0[:D], w1_0[D:D + 1], w1_0[D + 1:], jnp.tile(w2_0, (1, D)),
            w3_0[:D], w3_0[D:])
    wts1 = (iden, w1_1[:D], w1_1[D:D + 1], w1_1[D + 1:], jnp.tile(w2_1, (1, D)),
            w3_1[:D], w3_1[D:])

    out1, sess = _agg_a(h, g1, items3, maskr, wrow1, pos_1, wts0)
    finals = [
        _agg_bc(k, g1, g2[k], wrow2, pos_2, sess, out1, wrow1, pos_1,
                wts0, wts1)
        for k in range(KPART)]
    return jnp.concatenate(finals, axis=0)


# speedup vs baseline: 1.0828x; 1.0004x over previous
import functools

import jax
import jax.numpy as jnp
from jax import lax
from jax.experimental import pallas as pl
from jax.experimental.pallas import tpu as pltpu
from jax.experimental.pallas import tpu_sc as plsc

B, L, S, D, P = 128, 20, 10, 128, 16
M1 = L * S
M2 = L * S * S

LP = 2 * L
N1 = B * M1
NI = B * LP
N2 = B * M2

KPART = 4
BQ = B // KPART
NPART = BQ * M2

NC, NS = 2, 16
NW = NC * NS

GBA = 16
GBB = 8
GBC = 16


def _ring(tab_hbm, idx_v, out_hbm, base, npw, ch, bufs, sgs, sos, lag):
    nbuf = len(bufs)
    nchunk = npw // ch
    assert nchunk * ch == npw and nchunk % nbuf == 0 and 0 < lag < nbuf

    def start_gather(cc, b):
        ioff = pl.multiple_of(cc * ch, 8)
        pltpu.async_copy(
            tab_hbm.at[idx_v.at[pl.ds(ioff, ch)]], bufs[b], sgs[b])

    def wait_gather(b):
        pltpu.make_async_copy(
            tab_hbm.at[pl.ds(0, ch)], bufs[b], sgs[b]).wait()

    def start_out(cc, b):
        off = pl.multiple_of(base + cc * ch, 8)
        pltpu.async_copy(bufs[b], out_hbm.at[pl.ds(off, ch)], sos[b])

    def wait_out(b):
        pltpu.make_async_copy(
            bufs[b], out_hbm.at[pl.ds(0, ch)], sos[b]).wait()

    for c in range(lag):
        start_gather(c, c % nbuf)

    def step(i, carry):
        for b in range(nbuf):
            c = i * nbuf + b
            bn = (b + lag) % nbuf

            @pl.when(c + lag < nchunk)
            def _():
                @pl.when(c + lag >= nbuf)
                def _():
                    wait_out(bn)
                start_gather(c + lag, bn)

            wait_gather(b)
            start_out(c, b)
        return carry

    lax.fori_loop(0, nchunk // nbuf, step, 0)
    for b in range(nbuf):
        wait_out(b)


CH = 80
NBUF = 5


def _gather_quarter(idx2, table, kpart):
    mesh = plsc.VectorSubcoreMesh(core_axis_name="c", subcore_axis_name="s")
    npw = NPART // NW

    @functools.partial(
        pl.kernel,
        mesh=mesh,
        out_type=jax.ShapeDtypeStruct((NPART, D), jnp.float32),
        scratch_types=[
            pltpu.VMEM((npw,), jnp.int32),
        ] + [pltpu.VMEM((CH, D), jnp.float32)] * NBUF
          + [pltpu.SemaphoreType.DMA] * (2 * NBUF),
    )
    def k(idx_hbm, tab_hbm, out_hbm, idx_v, *rest):
        bufs = rest[:NBUF]
        sgs = rest[NBUF:2 * NBUF]
        sos = rest[2 * NBUF:]
        wid = lax.axis_index("s") * NC + lax.axis_index("c")
        base = wid * npw
        pltpu.sync_copy(idx_hbm.at[pl.ds(kpart * NPART + base, npw)], idx_v)
        _ring(tab_hbm, idx_v, out_hbm, base, npw, CH, bufs, sgs, sos, lag=3)

    return k(idx2, table)


def _gather_first(idx1, idxi, table):
    mesh = plsc.VectorSubcoreMesh(core_axis_name="c", subcore_axis_name="s")
    npw1 = N1 // NW
    npwi = NI // NW

    @functools.partial(
        pl.kernel,
        mesh=mesh,
        out_type=[
            jax.ShapeDtypeStruct((N1, D), jnp.float32),
            jax.ShapeDtypeStruct((NI, D), jnp.float32),
        ],
        scratch_types=[
            pltpu.VMEM((npw1,), jnp.int32),
            pltpu.VMEM((npwi,), jnp.int32),
        ] + [pltpu.VMEM((CH, D), jnp.float32)] * NBUF
          + [pltpu.SemaphoreType.DMA] * (2 * NBUF),
    )
    def k(idx1_hbm, idxi_hbm, tab_hbm, out1_hbm, outi_hbm,
          idx1_v, idxi_v, *rest):
        bufs = rest[:NBUF]
        sgs = rest[NBUF:2 * NBUF]
        sos = rest[2 * NBUF:]
        wid = lax.axis_index("s") * NC + lax.axis_index("c")
        pltpu.sync_copy(idx1_hbm.at[pl.ds(wid * npw1, npw1)], idx1_v)
        pltpu.sync_copy(idxi_hbm.at[pl.ds(wid * npwi, npwi)], idxi_v)
        _ring(tab_hbm, idx1_v, out1_hbm, wid * npw1, npw1, CH,
              bufs, sgs, sos, lag=3)
        _ring(tab_hbm, idxi_v, outi_hbm, wid * npwi, npwi, CH,
              bufs[:2], sgs[:2], sos[:2], lag=1)

    return k(idx1, idxi, table)


def _agg_math(neigh, self_v, wrows, pos, sess_blk, iden, w1h, w1w, w1p,
              w2m, w3a, w3b, gb, m):
    rows = gb * m * S
    groups = gb * m
    rb = m * S
    f32 = jnp.float32
    parts = []
    for g in range(gb):
        w1h_g = jnp.dot(iden * sess_blk[g:g + 1, :], w1h,
                        preferred_element_type=f32)
        parts.append(jnp.dot(neigh[g * rb:(g + 1) * rb], w1h_g,
                             preferred_element_type=f32))
    t = jnp.concatenate(parts, axis=0) if gb > 1 else parts[0]
    t = t + jnp.dot(pos, w1p, preferred_element_type=f32)
    wparts = [
        lax.dot_general(wrows[g], w1w, (((0,), (0,)), ((), ())),
                        preferred_element_type=f32)
        for g in range(gb)
    ]
    t = t + (jnp.concatenate(wparts, axis=0) if gb > 1 else wparts[0])
    t = jnp.maximum(t, 0.2 * t)
    logit = jnp.dot(t, w2m, preferred_element_type=f32)
    p = jnp.exp(logit - jnp.max(logit))
    num = jnp.sum((p * neigh).reshape(groups, S, D), axis=1)
    den = jnp.sum(p.reshape(groups, S, D), axis=1)
    agg = num / den
    out = jnp.dot(self_v, w3a, preferred_element_type=f32)
    out = out + jnp.dot(agg, w3b, preferred_element_type=f32)
    return jnp.maximum(out, 0.0)


def _agg_math_mb(neighs, self_v, wrows, pos, sess_blk, iden, w1h, w1w, w1p,
                 w2m, w3a, w3b, gb, m):
    rb = m * S
    f32 = jnp.float32
    aggs = []
    for g in range(gb):
        neigh = neighs[g]
        w1h_g = jnp.dot(iden * sess_blk[g:g + 1, :], w1h,
                        preferred_element_type=f32)
        t = jnp.dot(neigh, w1h_g, preferred_element_type=f32)
        t = t + jnp.dot(pos[g * rb:(g + 1) * rb], w1p,
                        preferred_element_type=f32)
        t = t + lax.dot_general(wrows[g], w1w, (((0,), (0,)), ((), ())),
                                preferred_element_type=f32)
        t = jnp.maximum(t, 0.2 * t)
        logit = jnp.dot(t, w2m, preferred_element_type=f32)
        p = jnp.exp(logit - jnp.max(logit))
        num = jnp.sum((p * neigh).reshape(m, S, D), axis=1)
        den = jnp.sum(p.reshape(m, S, D), axis=1)
        aggs.append(num / den)
    agg = jnp.concatenate(aggs, axis=0) if gb > 1 else aggs[0]
    out = jnp.dot(self_v, w3a, preferred_element_type=f32)
    out = out + jnp.dot(agg, w3b, preferred_element_type=f32)
    return jnp.maximum(out, 0.0)


_WSPECS = [
    pl.BlockSpec((D, D), lambda *a: (0, 0)),
    pl.BlockSpec((D, D), lambda *a: (0, 0)),
    pl.BlockSpec((1, D), lambda *a: (0, 0)),
    pl.BlockSpec((P, D), lambda *a: (0, 0)),
    pl.BlockSpec((D, D), lambda *a: (0, 0)),
    pl.BlockSpec((D, D), lambda *a: (0, 0)),
    pl.BlockSpec((D, D), lambda *a: (0, 0)),
]


def _agg_a(h, g1, items3, maskr, wrow1, pos_1, wts):
    def body(h_ref, *refs):
        neigh_refs = refs[:GBA]
        (it_ref, mask_ref, wr_ref, pos_ref,
         id_ref, w1h_ref, w1w_ref, w1p_ref, w2m_ref, w3a_ref, w3b_ref,
         out_ref, sess_ref) = refs[GBA:]
        f32 = jnp.float32
        items = it_ref[...].reshape(GBA * LP, D)
        mask = mask_ref[0]
        gid = lax.broadcasted_iota(jnp.int32, (GBA, GBA * LP), 0)
        rid = lax.broadcasted_iota(jnp.int32, (GBA, GBA * LP), 1) // LP
        mmat = jnp.where(gid == rid, mask, 0.0)
        sess_blk = jnp.dot(mmat, items, preferred_element_type=f32)
        sess_blk = sess_blk / jnp.sum(mmat, axis=1, keepdims=True)
        sess_ref[...] = sess_blk.reshape(GBA, 1, D)
        out = _agg_math_mb(
            [r[0] for r in neigh_refs],
            h_ref[...].reshape(GBA * L, D),
            wr_ref[...], pos_ref[...].reshape(GBA * M1, P), sess_blk,
            id_ref[...], w1h_ref[...], w1w_ref[...], w1p_ref[...],
            w2m_ref[...], w3a_ref[...], w3b_ref[...], GBA, L)
        out_ref[...] = out.reshape(GBA, L, D)

    def neigh_spec(i):
        return pl.BlockSpec((1, M1, D), lambda g: (g * GBA + i, 0, 0))

    return pl.pallas_call(
        body,
        grid=(B // GBA,),
        in_specs=[
            pl.BlockSpec((GBA, L, D), lambda g: (g, 0, 0)),
        ] + [neigh_spec(i) for i in range(GBA)] + [
            pl.BlockSpec((GBA, LP, D), lambda g: (g, 0, 0)),
            pl.BlockSpec((1, 1, GBA * LP), lambda g: (g, 0, 0)),
            pl.BlockSpec((GBA, 1, M1), lambda g: (g, 0, 0)),
            pl.BlockSpec((GBA, M1, P), lambda g: (g, 0, 0)),
        ] + _WSPECS,
        out_specs=[
            pl.BlockSpec((GBA, L, D), lambda g: (g, 0, 0)),
            pl.BlockSpec((GBA, 1, D), lambda g: (g, 0, 0)),
        ],
        out_shape=[
            jax.ShapeDtypeStruct((B, L, D), jnp.float32),
            jax.ShapeDtypeStruct((B, 1, D), jnp.float32),
        ],
    )(h, *([g1] * GBA), items3, maskr, wrow1, pos_1, *wts)


def _agg_bc(kpart, g1, g2k, wrow2, pos_2, sess, out1, wrow1, pos_1,
            wts0, wts1):
    k16 = kpart * (BQ // GBB)

    def body(self_ref, *refs):
        neigh_refs = refs[:GBB]
        (wr2_ref, pos2_ref, sess_ref, o1_ref, wr1_ref, pos1_ref,
         id_ref, w1h0, w1w0, w1p0, w2m0, w3a0, w3b0,
         w1h1, w1w1, w1p1, w2m1, w3a1, w3b1, out_ref) = refs[GBB:]
        sess_blk = sess_ref[...].reshape(GBB, D)
        out2 = _agg_math_mb(
            [r[0] for r in neigh_refs],
            self_ref[...].reshape(GBB * M1, D),
            wr2_ref[...], pos2_ref[...].reshape(GBB * M2, P), sess_blk,
            id_ref[...], w1h0[...], w1w0[...], w1p0[...],
            w2m0[...], w3a0[...], w3b0[...], GBB, M1)
        final = _agg_math_mb(
            [out2[g * M1:(g + 1) * M1] for g in range(GBB)],
            o1_ref[...].reshape(GBB * L, D),
            wr1_ref[...], pos1_ref[...].reshape(GBB * M1, P), sess_blk,
            id_ref[...], w1h1[...], w1w1[...], w1p1[...],
            w2m1[...], w3a1[...], w3b1[...], GBB, L)
        out_ref[...] = final.reshape(GBB, L, D)

    def neigh_spec(i):
        return pl.BlockSpec((1, M2, D), lambda g: (g * GBB + i, 0, 0))

    return pl.pallas_call(
        body,
        grid=(BQ // GBB,),
        in_specs=[
            pl.BlockSpec((GBB, M1, D), lambda g: (k16 + g, 0, 0)),
        ] + [neigh_spec(i) for i in range(GBB)] + [
            pl.BlockSpec((GBB, 1, M2), lambda g: (k16 + g, 0, 0)),
            pl.BlockSpec((GBB, M2, P), lambda g: (k16 + g, 0, 0)),
            pl.BlockSpec((GBB, 1, D), lambda g: (k16 + g, 0, 0)),
            pl.BlockSpec((GBB, L, D), lambda g: (k16 + g, 0, 0)),
            pl.BlockSpec((GBB, 1, M1), lambda g: (k16 + g, 0, 0)),
            pl.BlockSpec((GBB, M1, P), lambda g: (k16 + g, 0, 0)),
        ] + _WSPECS + _WSPECS[1:],
        out_specs=pl.BlockSpec((GBB, L, D), lambda g: (g, 0, 0)),
        out_shape=jax.ShapeDtypeStruct((BQ, L, D), jnp.float32),
    )(g1, *([g2k] * GBB), wrow2, pos_2, sess, out1, wrow1, pos_1,
      *wts0, *wts1[1:])


def kernel(h, neighbors_1, neighbors_2, weights_1, weights_2, pos_1, pos_2,
           item, mask_item, embedding, w1_0, w2_0, w3_0, w1_1, w2_1, w3_1):
    item_p = jnp.concatenate(
        [item, jnp.zeros((B, LP - L), dtype=item.dtype)], axis=1)
    idx1 = neighbors_1.reshape(-1).astype(jnp.int32)
    idxi = item_p.reshape(-1).astype(jnp.int32)
    idx2 = neighbors_2.reshape(-1).astype(jnp.int32)

    g2_0 = _gather_quarter(idx2, embedding, 0)
    g1_rows, item_rows = _gather_first(idx1, idxi, embedding)
    g2 = [g2_0.reshape(BQ, M2, D)] + [
        _gather_quarter(idx2, embedding, k).reshape(BQ, M2, D)
        for k in range(1, KPART)]
    g1 = g1_rows.reshape(B, M1, D)
    items3 = item_rows.reshape(B, LP, D)

    wrow1 = weights_1.reshape(B, 1, M1)
    wrow2 = weights_2.reshape(B, 1, M2)
    maskr = jnp.concatenate(
        [mask_item, jnp.zeros((B, LP - L), dtype=mask_item.dtype)],
        axis=1).reshape(B // GBA, 1, GBA * LP)

    iden = jnp.eye(D, dtype=jnp.float32)
    wts0 = (iden, w1_0[:D], w1_0[D:D + 1], w1_0[D + 1:], jnp.tile(w2_0, (1, D)),
            w3_0[:D], w3_0[D:])
    wts1 = (iden, w1_1[:D], w1_1[D:D + 1], w1_1[D + 1:], jnp.tile(w2_1, (1, D)),
            w3_1[:D], w3_1[D:])

    out1, sess = _agg_a(h, g1, items3, maskr, wrow1, pos_1, wts0)
    finals = [
        _agg_bc(k, g1, g2[k], wrow2, pos_2, sess, out1, wrow1, pos_1,
                wts0, wts1)
        for k in range(KPART)]
    return jnp.concatenate(finals, axis=0)
